# baseline (speedup 1.0000x reference)
import functools

import jax
import jax.numpy as jnp
from jax import lax
from jax.experimental import pallas as pl
from jax.experimental.pallas import tpu as pltpu
from jax.experimental.pallas import tpu_sc as plsc

_INFO = plsc.get_sparse_core_info()
_NC = _INFO.num_cores
_NS = _INFO.num_subcores
_NW = _NC * _NS
_L = _INFO.num_lanes


def _make_lookup(n, vocab, embed, chunk):
    assert n % (_NW * chunk) == 0 and chunk % _L == 0
    per_w = n // _NW
    n_chunks = per_w // chunk
    n_groups = chunk // _L
    mesh = plsc.VectorSubcoreMesh(core_axis_name="c", subcore_axis_name="s")

    @functools.partial(
        pl.kernel,
        out_type=jax.ShapeDtypeStruct((n * embed,), jnp.float32),
        mesh=mesh,
        scratch_types=[
            pltpu.VMEM((vocab * embed,), jnp.float32),
            pltpu.VMEM((2, chunk), jnp.int32),
            pltpu.VMEM((2, chunk * embed), jnp.float32),
            pltpu.SemaphoreType.DMA,
            pltpu.SemaphoreType.DMA,
            pltpu.SemaphoreType.DMA,
            pltpu.SemaphoreType.DMA,
        ],
        compiler_params=pltpu.CompilerParams(use_tc_tiling_on_sc=False,
                                             needs_layout_passes=False),
    )
    def lookup(day_hbm, table_hbm, out_hbm, table_v, idx_v, rows_v,
               i_sem0, i_sem1, o_sem0, o_sem1):
        wid = lax.axis_index("s") * _NC + lax.axis_index("c")
        w_base = wid * per_w
        i_sems = (i_sem0, i_sem1)
        o_sems = (o_sem0, o_sem1)

        pltpu.sync_copy(table_hbm, table_v)
        pltpu.sync_copy(day_hbm.at[pl.ds(w_base, chunk)], idx_v.at[0])
        lane = lax.iota(jnp.int32, _L)

        def chunk_body(g, carry):
            for b in (0, 1):

                @pl.when(g % 2 == b)
                def _():
                    nb = 1 - b
                    base = w_base + g * chunk

                    @pl.when(g + 1 < n_chunks)
                    def _():
                        pltpu.async_copy(
                            day_hbm.at[pl.ds(base + chunk, chunk)],
                            idx_v.at[nb], i_sems[nb])

                    @pl.when(g >= 2)
                    def _():
                        pltpu.make_async_copy(
                            rows_v.at[b],
                            out_hbm.at[pl.ds(base * embed, chunk * embed)],
                            o_sems[b]).wait()

                    def group_body(k, carry2):
                        dvec = idx_v[b, pl.ds(k * _L, _L)]
                        ga = dvec * embed
                        sa = lane * embed + k * (_L * embed)
                        for c in range(embed):
                            vals = plsc.load_gather(table_v, [ga + c])
                            plsc.store_scatter(rows_v.at[b], [sa + c], vals)
                        return carry2

                    lax.fori_loop(0, n_groups, group_body, 0)

                    pltpu.async_copy(
                        rows_v.at[b],
                        out_hbm.at[pl.ds(base * embed, chunk * embed)],
                        o_sems[b])

                    @pl.when(g + 1 < n_chunks)
                    def _():
                        pltpu.make_async_copy(
                            day_hbm.at[pl.ds(base + chunk, chunk)],
                            idx_v.at[nb], i_sems[nb]).wait()

            return carry

        lax.fori_loop(0, n_chunks, chunk_body, 0)

        for last in (n_chunks - 2, n_chunks - 1):
            pltpu.make_async_copy(
                rows_v.at[last % 2],
                out_hbm.at[pl.ds((w_base + last * chunk) * embed,
                                 chunk * embed)],
                o_sems[last % 2]).wait()

    return lookup


def kernel(day, table):
    batch, hist = day.shape
    vocab, embed = table.shape
    n = batch * hist
    day_flat = day.reshape(n).astype(jnp.int32)
    table_flat = table.reshape(vocab * embed)
    lookup = _make_lookup(n, vocab, embed, chunk=800)
    out = lookup(day_flat, table_flat)
    return out.reshape(batch, hist, embed)


# speedup vs baseline: 2.1948x; 2.1948x over previous
import functools

import jax
import jax.numpy as jnp
from jax import lax
from jax.experimental import pallas as pl
from jax.experimental.pallas import tpu as pltpu
from jax.experimental.pallas import tpu_sc as plsc

_INFO = plsc.get_sparse_core_info()
_NC = _INFO.num_cores
_NS = _INFO.num_subcores
_NW = _NC * _NS
_L = _INFO.num_lanes


def _make_lookup(n, vocab, embed, chunk):
    assert n % (_NW * chunk) == 0 and chunk % _L == 0
    per_w = n // _NW
    n_chunks = per_w // chunk
    n_groups = chunk // _L
    stride = embed + 1
    mesh = plsc.VectorSubcoreMesh(core_axis_name="c", subcore_axis_name="s")

    @functools.partial(
        pl.kernel,
        out_type=jax.ShapeDtypeStruct((n, embed), jnp.float32),
        mesh=mesh,
        scratch_types=[
            pltpu.VMEM((vocab * stride,), jnp.float32),
            pltpu.VMEM((2, chunk), jnp.int32),
            pltpu.VMEM((2, chunk, stride), jnp.float32),
            pltpu.SemaphoreType.DMA,
            pltpu.SemaphoreType.DMA,
            pltpu.SemaphoreType.DMA,
            pltpu.SemaphoreType.DMA,
        ],
        compiler_params=pltpu.CompilerParams(use_tc_tiling_on_sc=False,
                                             needs_layout_passes=False),
    )
    def lookup(day_hbm, table_hbm, out_hbm, table_v, idx_v, rows_v,
               i_sem0, i_sem1, o_sem0, o_sem1):
        wid = lax.axis_index("s") * _NC + lax.axis_index("c")
        w_base = wid * per_w
        i_sems = (i_sem0, i_sem1)
        o_sems = (o_sem0, o_sem1)

        pltpu.sync_copy(table_hbm, table_v)
        pltpu.sync_copy(day_hbm.at[pl.ds(w_base, chunk)], idx_v.at[0])
        lane = lax.iota(jnp.int32, _L)

        def chunk_body(g, carry):
            for b in (0, 1):

                @pl.when(g % 2 == b)
                def _():
                    nb = 1 - b
                    base = w_base + g * chunk

                    @pl.when(g + 1 < n_chunks)
                    def _():
                        pltpu.async_copy(
                            day_hbm.at[pl.ds(base + chunk, chunk)],
                            idx_v.at[nb], i_sems[nb])

                    @pl.when(g >= 2)
                    def _():
                        pltpu.make_async_copy(
                            rows_v.at[b, :, pl.ds(0, embed)],
                            out_hbm.at[pl.ds(base, chunk)],
                            o_sems[b]).wait()

                    def group_body(k, carry2):
                        dvec = idx_v[b, pl.ds(k * _L, _L)]
                        ga = dvec * stride
                        rvec = lane + k * _L
                        cv = jnp.zeros((_L,), jnp.int32)
                        for c in range(embed):
                            vals = plsc.load_gather(table_v, [ga + c])
                            plsc.store_scatter(rows_v.at[b], [rvec, cv + c],
                                               vals)
                        return carry2

                    lax.fori_loop(0, n_groups, group_body, 0)

                    pltpu.async_copy(
                        rows_v.at[b, :, pl.ds(0, embed)],
                        out_hbm.at[pl.ds(base, chunk)],
                        o_sems[b])

                    @pl.when(g + 1 < n_chunks)
                    def _():
                        pltpu.make_async_copy(
                            day_hbm.at[pl.ds(base + chunk, chunk)],
                            idx_v.at[nb], i_sems[nb]).wait()

            return carry

        lax.fori_loop(0, n_chunks, chunk_body, 0)

        for last in (n_chunks - 2, n_chunks - 1):
            pltpu.make_async_copy(
                rows_v.at[last % 2, :, pl.ds(0, embed)],
                out_hbm.at[pl.ds(w_base + last * chunk, chunk)],
                o_sems[last % 2]).wait()

    return lookup


def kernel(day, table):
    batch, hist = day.shape
    vocab, embed = table.shape
    n = batch * hist
    day_flat = day.reshape(n).astype(jnp.int32)
    table_padded = jnp.pad(table, ((0, 0), (0, 1))).reshape(-1)
    lookup = _make_lookup(n, vocab, embed, chunk=800)
    out = lookup(day_flat, table_padded)
    return out.reshape(batch, hist, embed)


# speedup vs baseline: 3.0869x; 1.4064x over previous
import functools

import jax
import jax.numpy as jnp
from jax import lax
from jax.experimental import pallas as pl
from jax.experimental.pallas import tpu as pltpu
from jax.experimental.pallas import tpu_sc as plsc

_INFO = plsc.get_sparse_core_info()
_NC = _INFO.num_cores
_NS = _INFO.num_subcores
_NW = _NC * _NS
_L = _INFO.num_lanes


def _make_lookup(n, vocab, embed, chunk):
    assert n % (_NW * chunk) == 0 and chunk % _L == 0
    per_w = n // _NW
    n_chunks = per_w // chunk
    n_groups = chunk // _L
    stride = embed + 1
    mesh = plsc.VectorSubcoreMesh(core_axis_name="c", subcore_axis_name="s")

    @functools.partial(
        pl.kernel,
        out_type=jax.ShapeDtypeStruct((n, embed), jnp.float32),
        mesh=mesh,
        scratch_types=[
            pltpu.VMEM((vocab * stride,), jnp.float32),
            pltpu.VMEM((2, chunk), jnp.int32),
            pltpu.VMEM((2, chunk, stride), jnp.float32),
            pltpu.SemaphoreType.DMA,
            pltpu.SemaphoreType.DMA,
            pltpu.SemaphoreType.DMA,
            pltpu.SemaphoreType.DMA,
        ],
        compiler_params=pltpu.CompilerParams(use_tc_tiling_on_sc=False,
                                             needs_layout_passes=False),
    )
    def lookup(day_hbm, table_hbm, out_hbm, table_v, idx_v, rows_v,
               i_sem0, i_sem1, o_sem0, o_sem1):
        wid = lax.axis_index("s") * _NC + lax.axis_index("c")
        w_base = wid * per_w
        i_sems = (i_sem0, i_sem1)
        o_sems = (o_sem0, o_sem1)

        pltpu.sync_copy(table_hbm, table_v)
        pltpu.sync_copy(day_hbm.at[pl.ds(w_base, chunk)], idx_v.at[0])
        lane = lax.iota(jnp.int32, _L)

        def chunk_body(g, carry):
            for b in (0, 1):

                @pl.when(g % 2 == b)
                def _():
                    nb = 1 - b
                    base = w_base + g * chunk

                    @pl.when(g + 1 < n_chunks)
                    def _():
                        pltpu.async_copy(
                            day_hbm.at[pl.ds(base + chunk, chunk)],
                            idx_v.at[nb], i_sems[nb])

                    @pl.when(g >= 2)
                    def _():
                        pltpu.make_async_copy(
                            rows_v.at[b, :, pl.ds(0, embed)],
                            out_hbm.at[pl.ds(base, chunk)],
                            o_sems[b]).wait()

                    @plsc.parallel_loop(0, n_groups, unroll=2)
                    def _(k):
                        dvec = idx_v[b, pl.ds(k * _L, _L)]
                        ga = dvec * stride
                        rvec = lane + k * _L
                        cv = jnp.zeros((_L,), jnp.int32)
                        for c in range(embed):
                            vals = plsc.load_gather(table_v, [ga + c])
                            plsc.store_scatter(rows_v.at[b], [rvec, cv + c],
                                               vals)

                    pltpu.async_copy(
                        rows_v.at[b, :, pl.ds(0, embed)],
                        out_hbm.at[pl.ds(base, chunk)],
                        o_sems[b])

                    @pl.when(g + 1 < n_chunks)
                    def _():
                        pltpu.make_async_copy(
                            day_hbm.at[pl.ds(base + chunk, chunk)],
                            idx_v.at[nb], i_sems[nb]).wait()

            return carry

        lax.fori_loop(0, n_chunks, chunk_body, 0)

        for last in (n_chunks - 2, n_chunks - 1):
            pltpu.make_async_copy(
                rows_v.at[last % 2, :, pl.ds(0, embed)],
                out_hbm.at[pl.ds(w_base + last * chunk, chunk)],
                o_sems[last % 2]).wait()

    return lookup


def kernel(day, table):
    batch, hist = day.shape
    vocab, embed = table.shape
    n = batch * hist
    day_flat = day.reshape(n).astype(jnp.int32)
    table_padded = jnp.pad(table, ((0, 0), (0, 1))).reshape(-1)
    lookup = _make_lookup(n, vocab, embed, chunk=800)
    out = lookup(day_flat, table_padded)
    return out.reshape(batch, hist, embed)


# speedup vs baseline: 3.0940x; 1.0023x over previous
import functools

import jax
import jax.numpy as jnp
from jax import lax
from jax.experimental import pallas as pl
from jax.experimental.pallas import tpu as pltpu
from jax.experimental.pallas import tpu_sc as plsc

_INFO = plsc.get_sparse_core_info()
_NC = _INFO.num_cores
_NS = _INFO.num_subcores
_NW = _NC * _NS
_L = _INFO.num_lanes


def _make_lookup(n, vocab, embed, chunk):
    assert n % (_NW * chunk) == 0 and chunk % _L == 0
    per_w = n // _NW
    n_chunks = per_w // chunk
    n_groups = chunk // _L
    stride = embed + 1
    mesh = plsc.VectorSubcoreMesh(core_axis_name="c", subcore_axis_name="s")

    @functools.partial(
        pl.kernel,
        out_type=jax.ShapeDtypeStruct((n, embed), jnp.float32),
        mesh=mesh,
        scratch_types=[
            pltpu.VMEM((vocab * stride,), jnp.float32),
            pltpu.VMEM((2, chunk), jnp.int32),
            pltpu.VMEM((2, chunk, stride), jnp.float32),
            pltpu.SemaphoreType.DMA,
            pltpu.SemaphoreType.DMA,
            pltpu.SemaphoreType.DMA,
            pltpu.SemaphoreType.DMA,
        ],
        compiler_params=pltpu.CompilerParams(use_tc_tiling_on_sc=False,
                                             needs_layout_passes=False),
    )
    def lookup(day_hbm, table_hbm, out_hbm, table_v, idx_v, rows_v,
               i_sem0, i_sem1, o_sem0, o_sem1):
        wid = lax.axis_index("s") * _NC + lax.axis_index("c")
        w_base = wid * per_w
        i_sems = (i_sem0, i_sem1)
        o_sems = (o_sem0, o_sem1)

        pltpu.sync_copy(table_hbm, table_v)
        pltpu.sync_copy(day_hbm.at[pl.ds(w_base, chunk)], idx_v.at[0])
        lane = lax.iota(jnp.int32, _L)

        def chunk_body(g, carry):
            for b in (0, 1):

                @pl.when(g % 2 == b)
                def _():
                    nb = 1 - b
                    base = w_base + g * chunk

                    @pl.when(g + 1 < n_chunks)
                    def _():
                        pltpu.async_copy(
                            day_hbm.at[pl.ds(base + chunk, chunk)],
                            idx_v.at[nb], i_sems[nb])

                    @pl.when(g >= 2)
                    def _():
                        pltpu.make_async_copy(
                            rows_v.at[b, :, pl.ds(0, embed)],
                            out_hbm.at[pl.ds(base, chunk)],
                            o_sems[b]).wait()

                    @plsc.parallel_loop(0, n_groups, unroll=1)
                    def _(k):
                        dvec = idx_v[b, pl.ds(k * _L, _L)]
                        ga = dvec * stride
                        rvec = lane + k * _L
                        cv = jnp.zeros((_L,), jnp.int32)
                        for c0 in range(0, embed, 8):
                            vals = [plsc.load_gather(table_v, [ga + (c0 + j)])
                                    for j in range(8)]
                            for j in range(8):
                                plsc.store_scatter(
                                    rows_v.at[b], [rvec, cv + (c0 + j)],
                                    vals[j])

                    pltpu.async_copy(
                        rows_v.at[b, :, pl.ds(0, embed)],
                        out_hbm.at[pl.ds(base, chunk)],
                        o_sems[b])

                    @pl.when(g + 1 < n_chunks)
                    def _():
                        pltpu.make_async_copy(
                            day_hbm.at[pl.ds(base + chunk, chunk)],
                            idx_v.at[nb], i_sems[nb]).wait()

            return carry

        lax.fori_loop(0, n_chunks, chunk_body, 0)

        for last in (n_chunks - 2, n_chunks - 1):
            pltpu.make_async_copy(
                rows_v.at[last % 2, :, pl.ds(0, embed)],
                out_hbm.at[pl.ds(w_base + last * chunk, chunk)],
                o_sems[last % 2]).wait()

    return lookup


def kernel(day, table):
    batch, hist = day.shape
    vocab, embed = table.shape
    n = batch * hist
    day_flat = day.reshape(n).astype(jnp.int32)
    table_padded = jnp.pad(table, ((0, 0), (0, 1))).reshape(-1)
    lookup = _make_lookup(n, vocab, embed, chunk=800)
    out = lookup(day_flat, table_padded)
    return out.reshape(batch, hist, embed)


# speedup vs baseline: 3.6454x; 1.1782x over previous
import functools

import jax
import jax.numpy as jnp
from jax import lax
from jax.experimental import pallas as pl
from jax.experimental.pallas import tpu as pltpu
from jax.experimental.pallas import tpu_sc as plsc

_INFO = plsc.get_sparse_core_info()
_NC = _INFO.num_cores
_NS = _INFO.num_subcores
_NW = _NC * _NS
_L = _INFO.num_lanes


def _make_lookup(n, vocab, embed, chunk):
    assert n % (_NW * chunk) == 0 and chunk % _L == 0
    per_w = n // _NW
    n_chunks = per_w // chunk
    n_groups = chunk // _L
    stride = embed + 1
    mesh = plsc.VectorSubcoreMesh(core_axis_name="c", subcore_axis_name="s")

    @functools.partial(
        pl.kernel,
        out_type=jax.ShapeDtypeStruct((n, embed), jnp.float32),
        mesh=mesh,
        scratch_types=[
            pltpu.VMEM((vocab * (embed + _L),), jnp.float32),
            pltpu.VMEM((2, chunk), jnp.int32),
            pltpu.VMEM((2, chunk, embed), jnp.float32),
            pltpu.SemaphoreType.DMA,
            pltpu.SemaphoreType.DMA,
            pltpu.SemaphoreType.DMA,
            pltpu.SemaphoreType.DMA,
        ],
        compiler_params=pltpu.CompilerParams(use_tc_tiling_on_sc=False,
                                             needs_layout_passes=False),
    )
    def lookup(day_hbm, table_hbm, out_hbm, table_v, idx_v, rows_v,
               i_sem0, i_sem1, o_sem0, o_sem1):
        wid = lax.axis_index("s") * _NC + lax.axis_index("c")
        w_base = wid * per_w
        i_sems = (i_sem0, i_sem1)
        o_sems = (o_sem0, o_sem1)

        pltpu.sync_copy(table_hbm, table_v)
        pltpu.sync_copy(day_hbm.at[pl.ds(w_base, chunk)], idx_v.at[0])
        lane = lax.iota(jnp.int32, _L)

        def chunk_body(g, carry):
            for b in (0, 1):

                @pl.when(g % 2 == b)
                def _():
                    nb = 1 - b
                    base = w_base + g * chunk

                    @pl.when(g + 1 < n_chunks)
                    def _():
                        pltpu.async_copy(
                            day_hbm.at[pl.ds(base + chunk, chunk)],
                            idx_v.at[nb], i_sems[nb])

                    @pl.when(g >= 2)
                    def _():
                        pltpu.make_async_copy(
                            rows_v.at[b],
                            out_hbm.at[pl.ds(base, chunk)],
                            o_sems[b]).wait()

                    @plsc.parallel_loop(0, n_groups, unroll=1)
                    def _(k):
                        dvec = idx_v[b, pl.ds(k * _L, _L)]
                        ga = dvec * (embed + _L) + lane
                        rvec = lane + k * _L
                        for c in range(embed):
                            vals = plsc.load_gather(table_v, [ga + c])
                            w = (lane + c) & (embed - 1)
                            plsc.store_scatter(rows_v.at[b], [rvec, w],
                                               vals)

                    pltpu.async_copy(
                        rows_v.at[b],
                        out_hbm.at[pl.ds(base, chunk)],
                        o_sems[b])

                    @pl.when(g + 1 < n_chunks)
                    def _():
                        pltpu.make_async_copy(
                            day_hbm.at[pl.ds(base + chunk, chunk)],
                            idx_v.at[nb], i_sems[nb]).wait()

            return carry

        lax.fori_loop(0, n_chunks, chunk_body, 0)

        for last in (n_chunks - 2, n_chunks - 1):
            pltpu.make_async_copy(
                rows_v.at[last % 2],
                out_hbm.at[pl.ds(w_base + last * chunk, chunk)],
                o_sems[last % 2]).wait()

    return lookup


def kernel(day, table):
    batch, hist = day.shape
    vocab, embed = table.shape
    n = batch * hist
    day_flat = day.reshape(n).astype(jnp.int32)
    table_ext = jnp.concatenate([table, table[:, :_L]], axis=1).reshape(-1)
    lookup = _make_lookup(n, vocab, embed, chunk=800)
    out = lookup(day_flat, table_ext)
    return out.reshape(batch, hist, embed)


# speedup vs baseline: 3.9868x; 1.0937x over previous
import functools

import jax
import jax.numpy as jnp
from jax import lax
from jax.experimental import pallas as pl
from jax.experimental.pallas import tpu as pltpu
from jax.experimental.pallas import tpu_sc as plsc

_INFO = plsc.get_sparse_core_info()
_NC = _INFO.num_cores
_NS = _INFO.num_subcores
_NW = _NC * _NS
_L = _INFO.num_lanes


def _make_lookup(n, vocab, embed, chunk, split):
    assert n % (_NW * chunk) == 0 and chunk % _L == 0 and split % _L == 0
    per_w = n // _NW
    n_chunks = per_w // chunk
    n_groups = chunk // _L
    s_groups = split // _L
    ext = embed + _L
    mesh = plsc.VectorSubcoreMesh(core_axis_name="c", subcore_axis_name="s")

    @functools.partial(
        pl.kernel,
        out_type=jax.ShapeDtypeStruct((n, embed), jnp.float32),
        mesh=mesh,
        scratch_types=[
            pltpu.VMEM_SHARED((vocab, embed), jnp.float32),
            pltpu.VMEM((vocab * ext,), jnp.float32),
            pltpu.VMEM((2, chunk), jnp.int32),
            pltpu.VMEM((2, chunk, embed), jnp.float32),
            pltpu.SemaphoreType.DMA,
            pltpu.SemaphoreType.DMA,
            pltpu.SemaphoreType.DMA,
            pltpu.SemaphoreType.DMA,
            pltpu.SemaphoreType.DMA,
            pltpu.SemaphoreType.DMA,
        ],
        compiler_params=pltpu.CompilerParams(use_tc_tiling_on_sc=False,
                                             needs_layout_passes=False),
    )
    def lookup(day_hbm, table_hbm, table_ext_hbm, out_hbm,
               table_sh, table_v, idx_v, rows_v,
               i_sem0, i_sem1, o_sem0, o_sem1, s_sem0, s_sem1):
        wid = lax.axis_index("s") * _NC + lax.axis_index("c")
        w_base = wid * per_w
        i_sems = (i_sem0, i_sem1)
        o_sems = (o_sem0, o_sem1)
        s_sems = (s_sem0, s_sem1)

        pltpu.sync_copy(table_hbm, table_sh)
        pltpu.sync_copy(table_ext_hbm, table_v)
        pltpu.sync_copy(day_hbm.at[pl.ds(w_base, chunk)], idx_v.at[0])
        lane = lax.iota(jnp.int32, _L)

        def chunk_body(g, carry):
            for b in (0, 1):

                @pl.when(g % 2 == b)
                def _():
                    nb = 1 - b
                    base = w_base + g * chunk

                    @pl.when(g + 1 < n_chunks)
                    def _():
                        pltpu.async_copy(
                            day_hbm.at[pl.ds(base + chunk, chunk)],
                            idx_v.at[nb], i_sems[nb])

                    @pl.when(g >= 2)
                    def _():
                        pltpu.make_async_copy(
                            rows_v.at[b],
                            out_hbm.at[pl.ds(base, chunk)],
                            o_sems[b]).wait()

                    pltpu.async_copy(
                        table_sh.at[idx_v.at[b, pl.ds(0, split)]],
                        rows_v.at[b, pl.ds(0, split)], s_sems[b])

                    @plsc.parallel_loop(s_groups, n_groups, unroll=1)
                    def _(k):
                        dvec = idx_v[b, pl.ds(k * _L, _L)]
                        ga = dvec * ext + lane
                        rvec = lane + k * _L
                        for c in range(embed):
                            vals = plsc.load_gather(table_v, [ga + c])
                            w = (lane + c) & (embed - 1)
                            plsc.store_scatter(rows_v.at[b], [rvec, w],
                                               vals)

                    pltpu.make_async_copy(
                        table_sh.at[idx_v.at[b, pl.ds(0, split)]],
                        rows_v.at[b, pl.ds(0, split)], s_sems[b]).wait()

                    pltpu.async_copy(
                        rows_v.at[b],
                        out_hbm.at[pl.ds(base, chunk)],
                        o_sems[b])

                    @pl.when(g + 1 < n_chunks)
                    def _():
                        pltpu.make_async_copy(
                            day_hbm.at[pl.ds(base + chunk, chunk)],
                            idx_v.at[nb], i_sems[nb]).wait()

            return carry

        lax.fori_loop(0, n_chunks, chunk_body, 0)

        for last in (n_chunks - 2, n_chunks - 1):
            pltpu.make_async_copy(
                rows_v.at[last % 2],
                out_hbm.at[pl.ds(w_base + last * chunk, chunk)],
                o_sems[last % 2]).wait()

    return lookup


def kernel(day, table):
    batch, hist = day.shape
    vocab, embed = table.shape
    n = batch * hist
    day_flat = day.reshape(n).astype(jnp.int32)
    table_ext = jnp.concatenate([table, table[:, :_L]], axis=1).reshape(-1)
    lookup = _make_lookup(n, vocab, embed, chunk=800, split=400)
    out = lookup(day_flat, table, table_ext)
    return out.reshape(batch, hist, embed)
